# baseline (speedup 1.0000x reference)
import functools

import jax
import jax.numpy as jnp
from jax import lax
from jax.experimental import pallas as pl
from jax.experimental.pallas import tpu as pltpu
from jax.experimental.pallas import tpu_sc as plsc

DIM = 64
MAXPOS = 200
LANE = 16
NCH = DIM // LANE
SUB = 128
NSUB = 8
K = SUB * NSUB
EPS = 1e-5


def _fast_rsqrt(v):
    bits = lax.bitcast_convert_type(v, jnp.int32)
    y = lax.bitcast_convert_type(jnp.int32(0x5F3759DF) - (bits >> 1), jnp.float32)
    for _ in range(3):
        y = y * (1.5 - 0.5 * v * y * y)
    return y


@functools.lru_cache(maxsize=None)
def _build(N):
    info = plsc.get_sparse_core_info()
    nc, ns = info.num_cores, info.num_subcores
    nw = nc * ns
    tok_w = N // nw
    nchunk = tok_w // K
    assert tok_w % K == 0 and N % (nw * SUB) == 0

    mesh = plsc.VectorSubcoreMesh(core_axis_name="c", subcore_axis_name="s")

    @functools.partial(
        pl.kernel,
        mesh=mesh,
        compiler_params=pltpu.CompilerParams(use_tc_tiling_on_sc=False),
        out_type=jax.ShapeDtypeStruct((N, DIM), jnp.float32),
        scratch_types=[
            pltpu.VMEM((NSUB, SUB), jnp.int32),
            pltpu.VMEM((K, DIM), jnp.float32),
            pltpu.VMEM((MAXPOS, DIM), jnp.float32),
            pltpu.VMEM((DIM,), jnp.float32),
            pltpu.VMEM((DIM,), jnp.float32),
            pltpu.SemaphoreType.DMA,
        ],
    )
    def emb(x_hbm, wt_hbm, pos_hbm, g_hbm, b_hbm, out_hbm,
            idx_v, rows_v, pos_v, g_v, b_v, sem):
        wid = lax.axis_index("s") * nc + lax.axis_index("c")
        base = wid * tok_w
        pltpu.sync_copy(pos_hbm, pos_v)
        pltpu.sync_copy(g_hbm, g_v)
        pltpu.sync_copy(b_hbm, b_v)
        gs = [g_v[pl.ds(c * LANE, LANE)] for c in range(NCH)]
        bs = [b_v[pl.ds(c * LANE, LANE)] for c in range(NCH)]
        lanes = lax.iota(jnp.int32, LANE)
        perms = [lanes ^ sh for sh in (8, 4, 2, 1)]

        def lane_sum(v):
            for p in perms:
                v = v + v.at[p].get(mode="promise_in_bounds")
            return v

        def chunk(ci, carry):
            off = pl.multiple_of(base + ci * K, K)
            pltpu.sync_copy(x_hbm.at[pl.ds(pl.multiple_of(off // SUB, NSUB), NSUB)], idx_v)
            copies = [
                pltpu.async_copy(wt_hbm.at[idx_v.at[g]],
                                 rows_v.at[pl.ds(g * SUB, SUB)], sem)
                for g in range(NSUB)
            ]
            for c in copies:
                c.wait()

            def tok(j, _):
                p = lax.rem(off + j, MAXPOS)
                hs = [rows_v[j, pl.ds(c * LANE, LANE)]
                      + pos_v[p, pl.ds(c * LANE, LANE)] for c in range(NCH)]
                tot = (hs[0] + hs[1]) + (hs[2] + hs[3])
                mean = lane_sum(tot) * (1.0 / DIM)
                d = [h - mean for h in hs]
                q = (d[0] * d[0] + d[1] * d[1]) + (d[2] * d[2] + d[3] * d[3])
                var = lane_sum(q) * (1.0 / DIM)
                rstd = _fast_rsqrt(var + EPS)
                for c in range(NCH):
                    rows_v[j, pl.ds(c * LANE, LANE)] = d[c] * rstd * gs[c] + bs[c]
                return 0

            lax.fori_loop(0, K, tok, 0)
            pltpu.sync_copy(rows_v, out_hbm.at[pl.ds(off, K)])
            return carry

        lax.fori_loop(0, nchunk, chunk, 0)

    return emb


def kernel(x, word_table, pos_table, gamma, beta):
    b, s = x.shape
    n = b * s
    x2 = x.reshape(n // SUB, SUB)
    out = _build(n)(x2, word_table, pos_table, gamma, beta)
    return out.reshape(b, s, DIM)


# speedup vs baseline: 1.0268x; 1.0268x over previous
import functools

import jax
import jax.numpy as jnp
from jax import lax
from jax.experimental import pallas as pl
from jax.experimental.pallas import tpu as pltpu
from jax.experimental.pallas import tpu_sc as plsc

DIM = 64
MAXPOS = 200
LANE = 16
NCH = DIM // LANE
SUB = 64
NSUB = 8
K = SUB * NSUB
UNROLL = 4
EPS = 1e-5


def _fast_rsqrt(v):
    bits = lax.bitcast_convert_type(v, jnp.int32)
    y = lax.bitcast_convert_type(jnp.int32(0x5F3759DF) - (bits >> 1), jnp.float32)
    for _ in range(3):
        y = y * (1.5 - 0.5 * v * y * y)
    return y


@functools.lru_cache(maxsize=None)
def _build(N):
    info = plsc.get_sparse_core_info()
    nc, ns = info.num_cores, info.num_subcores
    nw = nc * ns
    tok_w = N // nw
    nchunk = tok_w // K
    assert tok_w % K == 0 and nchunk % 2 == 0 and N % (nw * SUB) == 0

    mesh = plsc.VectorSubcoreMesh(core_axis_name="c", subcore_axis_name="s")

    @functools.partial(
        pl.kernel,
        mesh=mesh,
        compiler_params=pltpu.CompilerParams(use_tc_tiling_on_sc=False),
        out_type=jax.ShapeDtypeStruct((N, DIM), jnp.float32),
        scratch_types=[
            pltpu.VMEM((2, NSUB, SUB), jnp.int32),
            pltpu.VMEM((K, DIM), jnp.float32),
            pltpu.VMEM((K, DIM), jnp.float32),
            pltpu.VMEM((MAXPOS, DIM), jnp.float32),
            pltpu.VMEM((DIM,), jnp.float32),
            pltpu.VMEM((DIM,), jnp.float32),
            pltpu.SemaphoreType.DMA,
            pltpu.SemaphoreType.DMA,
        ],
    )
    def emb(x_hbm, wt_hbm, pos_hbm, g_hbm, b_hbm, out_hbm,
            idx_v, rows0_v, rows1_v, pos_v, g_v, b_v, sem0, sem1):
        wid = lax.axis_index("s") * nc + lax.axis_index("c")
        base = wid * tok_w
        pltpu.sync_copy(pos_hbm, pos_v)
        pltpu.sync_copy(g_hbm, g_v)
        pltpu.sync_copy(b_hbm, b_v)
        gs = [g_v[pl.ds(c * LANE, LANE)] for c in range(NCH)]
        bs = [b_v[pl.ds(c * LANE, LANE)] for c in range(NCH)]
        lanes = lax.iota(jnp.int32, LANE)
        perms = [lanes ^ sh for sh in (8, 4, 2, 1)]
        bufs = (rows0_v, rows1_v)
        sems = (sem0, sem1)

        def lane_sum(v):
            for p in perms:
                v = v + v.at[p].get(mode="promise_in_bounds")
            return v

        def gather_copies(ci, par):
            return [
                pltpu.make_async_copy(wt_hbm.at[idx_v.at[par].at[g]],
                                      bufs[par].at[pl.ds(g * SUB, SUB)],
                                      sems[par])
                for g in range(NSUB)
            ]

        def fire(ci, par):
            off = pl.multiple_of(base + ci * K, K)
            pltpu.sync_copy(
                x_hbm.at[pl.ds(pl.multiple_of(off // SUB, NSUB), NSUB)],
                idx_v.at[par])
            for c in gather_copies(ci, par):
                c.start()

        def drain(ci, par):
            for c in gather_copies(ci, par):
                c.wait()

        def compute_and_flush(ci, par):
            off = pl.multiple_of(base + ci * K, K)
            rows_v = bufs[par]

            def tok(t, _):
                jb = t * UNROLL
                for u in range(UNROLL):
                    j = jb + u
                    p = lax.rem(off + j, MAXPOS)
                    hs = [rows_v[j, pl.ds(c * LANE, LANE)]
                          + pos_v[p, pl.ds(c * LANE, LANE)] for c in range(NCH)]
                    tot = (hs[0] + hs[1]) + (hs[2] + hs[3])
                    mean = lane_sum(tot) * (1.0 / DIM)
                    d = [h - mean for h in hs]
                    q = (d[0] * d[0] + d[1] * d[1]) + (d[2] * d[2] + d[3] * d[3])
                    var = lane_sum(q) * (1.0 / DIM)
                    rstd = _fast_rsqrt(var + EPS)
                    for c in range(NCH):
                        rows_v[j, pl.ds(c * LANE, LANE)] = d[c] * rstd * gs[c] + bs[c]
                return 0

            lax.fori_loop(0, K // UNROLL, tok, 0)
            pltpu.sync_copy(rows_v, out_hbm.at[pl.ds(off, K)])

        fire(0, 0)

        def pair(i, _):
            for h in range(2):
                ci = 2 * i + h
                fire(ci + 1, 1 - h)
                drain(ci, h)
                compute_and_flush(ci, h)
            return 0

        lax.fori_loop(0, (nchunk - 2) // 2, pair, 0)
        fire(nchunk - 1, 1)
        drain(nchunk - 2, 0)
        compute_and_flush(nchunk - 2, 0)
        drain(nchunk - 1, 1)
        compute_and_flush(nchunk - 1, 1)

    return emb


def kernel(x, word_table, pos_table, gamma, beta):
    b, s = x.shape
    n = b * s
    x2 = x.reshape(n // SUB, SUB)
    out = _build(n)(x2, word_table, pos_table, gamma, beta)
    return out.reshape(b, s, DIM)


# speedup vs baseline: 1.7747x; 1.7284x over previous
import functools

import jax
import jax.numpy as jnp
from jax import lax
from jax.experimental import pallas as pl
from jax.experimental.pallas import tpu as pltpu
from jax.experimental.pallas import tpu_sc as plsc

DIM = 64
MAXPOS = 200
LANE = 16
NCH = DIM // LANE
SUB = 64
NSUB = 8
K = SUB * NSUB
UNROLL = 4
EPS = 1e-5


def _fast_rsqrt(v):
    bits = lax.bitcast_convert_type(v, jnp.int32)
    y = lax.bitcast_convert_type(jnp.int32(0x5F3759DF) - (bits >> 1), jnp.float32)
    for _ in range(3):
        y = y * (1.5 - 0.5 * v * y * y)
    return y


@functools.lru_cache(maxsize=None)
def _build(N):
    info = plsc.get_sparse_core_info()
    nc, ns = info.num_cores, info.num_subcores
    nw = nc * ns
    tok_w = N // nw
    nchunk = tok_w // K
    assert tok_w % K == 0 and nchunk % 2 == 0 and N % (nw * SUB) == 0

    mesh = plsc.VectorSubcoreMesh(core_axis_name="c", subcore_axis_name="s")

    @functools.partial(
        pl.kernel,
        mesh=mesh,
        compiler_params=pltpu.CompilerParams(use_tc_tiling_on_sc=False),
        out_type=jax.ShapeDtypeStruct((N, DIM), jnp.float32),
        scratch_types=[
            pltpu.VMEM((2, NSUB, SUB), jnp.int32),
            pltpu.VMEM((K, DIM), jnp.float32),
            pltpu.VMEM((K, DIM), jnp.float32),
            pltpu.VMEM((MAXPOS, DIM), jnp.float32),
            pltpu.VMEM((DIM,), jnp.float32),
            pltpu.VMEM((DIM,), jnp.float32),
            pltpu.SemaphoreType.DMA,
            pltpu.SemaphoreType.DMA,
        ],
    )
    def emb(x_hbm, wt_hbm, pos_hbm, g_hbm, b_hbm, out_hbm,
            idx_v, rows0_v, rows1_v, pos_v, g_v, b_v, sem0, sem1):
        wid = lax.axis_index("s") * nc + lax.axis_index("c")
        base = wid * tok_w
        pltpu.sync_copy(pos_hbm, pos_v)
        pltpu.sync_copy(g_hbm, g_v)
        pltpu.sync_copy(b_hbm, b_v)
        gs = [g_v[pl.ds(c * LANE, LANE)] for c in range(NCH)]
        bs = [b_v[pl.ds(c * LANE, LANE)] for c in range(NCH)]
        lanes = lax.iota(jnp.int32, LANE)
        perms = [lanes ^ sh for sh in (8, 4, 2, 1)]
        bufs = (rows0_v, rows1_v)
        sems = (sem0, sem1)

        def lane_sum(v):
            for p in perms:
                v = v + v.at[p].get(mode="promise_in_bounds")
            return v

        def gather_copies(ci, par):
            return [
                pltpu.make_async_copy(wt_hbm.at[idx_v.at[par].at[g]],
                                      bufs[par].at[pl.ds(g * SUB, SUB)],
                                      sems[par])
                for g in range(NSUB)
            ]

        def fire(ci, par):
            off = pl.multiple_of(base + ci * K, K)
            pltpu.sync_copy(
                x_hbm.at[pl.ds(pl.multiple_of(off // SUB, NSUB), NSUB)],
                idx_v.at[par])
            for c in gather_copies(ci, par):
                c.start()

        def drain(ci, par):
            for c in gather_copies(ci, par):
                c.wait()

        def compute_and_flush(ci, par):
            off = pl.multiple_of(base + ci * K, K)
            rows_v = bufs[par]

            @plsc.parallel_loop(0, K, 1, unroll=UNROLL)
            def tok(j):
                p = lax.rem(off + j, MAXPOS)
                hs = [rows_v[j, pl.ds(c * LANE, LANE)]
                      + pos_v[p, pl.ds(c * LANE, LANE)] for c in range(NCH)]
                tot = (hs[0] + hs[1]) + (hs[2] + hs[3])
                mean = lane_sum(tot) * (1.0 / DIM)
                d = [h - mean for h in hs]
                q = (d[0] * d[0] + d[1] * d[1]) + (d[2] * d[2] + d[3] * d[3])
                var = lane_sum(q) * (1.0 / DIM)
                rstd = _fast_rsqrt(var + EPS)
                for c in range(NCH):
                    rows_v[j, pl.ds(c * LANE, LANE)] = d[c] * rstd * gs[c] + bs[c]
            pltpu.sync_copy(rows_v, out_hbm.at[pl.ds(off, K)])

        fire(0, 0)

        def pair(i, _):
            for h in range(2):
                ci = 2 * i + h
                fire(ci + 1, 1 - h)
                drain(ci, h)
                compute_and_flush(ci, h)
            return 0

        lax.fori_loop(0, (nchunk - 2) // 2, pair, 0)
        fire(nchunk - 1, 1)
        drain(nchunk - 2, 0)
        compute_and_flush(nchunk - 2, 0)
        drain(nchunk - 1, 1)
        compute_and_flush(nchunk - 1, 1)

    return emb


def kernel(x, word_table, pos_table, gamma, beta):
    b, s = x.shape
    n = b * s
    x2 = x.reshape(n // SUB, SUB)
    out = _build(n)(x2, word_table, pos_table, gamma, beta)
    return out.reshape(b, s, DIM)
